# baseline (speedup 1.0000x reference)
import jax
import jax.numpy as jnp
from jax import lax
from jax.experimental import pallas as pl
from jax.experimental.pallas import tpu as pltpu
from jax.experimental.pallas import tpu_sc as plsc

D = 768
E = 16
DFF = 4 * D
T = 2048
BT = 256
MAXB = 24
PADT = MAXB * BT
TC = 256
NCH = T // TC
NC, NS = 2, 16
NW = NC * NS
RPW = T // NW
NCHK = 4
CHK = RPW // NCHK


def _gate_body(x1_ref, x2_ref, x3_ref, x4_ref, wg_ref, bg_ref,
               gate_ref, dest_ref, meta_ref):
    wg = wg_ref[...]
    logits = jnp.concatenate(
        [jnp.dot(xr[...], wg, preferred_element_type=jnp.float32)
         for xr in (x1_ref, x2_ref, x3_ref, x4_ref)], axis=0) + bg_ref[...]
    lane = lax.broadcasted_iota(jnp.int32, (T, E), 1)
    maxv = jnp.max(logits, axis=1, keepdims=True)
    idx = jnp.min(jnp.where(logits == maxv, lane, E), axis=1, keepdims=True)
    oh_i = (lane == idx).astype(jnp.int32)
    oh_f = oh_i.astype(jnp.float32)
    gate_ref[...] = logits + (oh_f - logits)

    c = oh_i
    k = 1
    while k < T:
        c = c + jnp.concatenate(
            [jnp.zeros((k, E), jnp.int32), c[: T - k, :]], axis=0)
        k *= 2
    counts = c[T - 1 : T, :]
    nb = (counts + (BT - 1)) // BT
    be = nb
    k = 1
    while k < E:
        be = be + jnp.concatenate(
            [jnp.zeros((1, k), jnp.int32), be[:, : E - k]], axis=1)
        k *= 2
    pad_off = (be - nb) * BT
    total = be[:, E - 1 : E]

    biota = lax.broadcasted_iota(jnp.int32, (MAXB, E), 0)
    eob = jnp.sum((biota >= be).astype(jnp.int32), axis=1, keepdims=True)
    eob = jnp.minimum(eob, E - 1)
    meta_ref[...] = jnp.concatenate(
        [eob, total, jnp.zeros((7, 1), jnp.int32)], axis=0)

    dest_ref[...] = jnp.sum(oh_i * (pad_off + c - 1), axis=1, keepdims=True)


_gate_call = pl.pallas_call(
    _gate_body,
    grid=(1,),
    in_specs=(
        [pl.BlockSpec((T // 4, D), (lambda k: (lambda i: (k, 0)))(k))
         for k in range(4)]
        + [pl.BlockSpec((D, E), lambda i: (0, 0)),
           pl.BlockSpec((1, E), lambda i: (0, 0))]
    ),
    out_specs=[
        pl.BlockSpec((T, E), lambda i: (0, 0)),
        pl.BlockSpec((T, 1), lambda i: (0, 0)),
        pl.BlockSpec((MAXB + 8, 1), lambda i: (0, 0)),
    ],
    out_shape=(
        jax.ShapeDtypeStruct((T, E), jnp.float32),
        jax.ShapeDtypeStruct((T, 1), jnp.int32),
        jax.ShapeDtypeStruct((MAXB + 8, 1), jnp.int32),
    ),
)


def _ffn_body(meta_ref, x_ref, wu1_ref, wu2_ref, bu_ref, wd1_ref, wd2_ref,
              bd_ref, o_ref):
    b = pl.program_id(0)

    @pl.when(b < meta_ref[MAXB])
    def _():
        xb = x_ref[...].astype(jnp.bfloat16)
        h = (jnp.dot(xb[:, : D // 2], wu1_ref[0, 0].astype(jnp.bfloat16),
                     preferred_element_type=jnp.float32)
             + jnp.dot(xb[:, D // 2 :], wu2_ref[0, 0].astype(jnp.bfloat16),
                       preferred_element_type=jnp.float32)
             + bu_ref[0])
        h = jnp.maximum(h, 0.0).astype(jnp.bfloat16)
        o_ref[...] = (
            jnp.dot(h[:, : DFF // 2], wd1_ref[0, 0].astype(jnp.bfloat16),
                    preferred_element_type=jnp.float32)
            + jnp.dot(h[:, DFF // 2 :], wd2_ref[0, 0].astype(jnp.bfloat16),
                      preferred_element_type=jnp.float32)
            + bd_ref[0])


def _bmap(b, m):
    return (jnp.minimum(b, m[MAXB] - 1), 0)


def _emap3(b, m):
    return (m[jnp.minimum(b, m[MAXB] - 1)], 0, 0)


def _emap4a(b, m):
    return (m[jnp.minimum(b, m[MAXB] - 1)], 0, 0, 0)


def _emap4b(b, m):
    return (m[jnp.minimum(b, m[MAXB] - 1)], 1, 0, 0)


_ffn_grid_spec = pltpu.PrefetchScalarGridSpec(
    num_scalar_prefetch=1,
    grid=(MAXB,),
    in_specs=[
        pl.BlockSpec((BT, D), _bmap),
        pl.BlockSpec((1, 1, D // 2, DFF), _emap4a),
        pl.BlockSpec((1, 1, D // 2, DFF), _emap4b),
        pl.BlockSpec((1, 1, DFF), _emap3),
        pl.BlockSpec((1, 1, DFF // 2, D), _emap4a),
        pl.BlockSpec((1, 1, DFF // 2, D), _emap4b),
        pl.BlockSpec((1, 1, D), _emap3),
    ],
    out_specs=pl.BlockSpec((BT, D), _bmap),
)

_ffn_call = pl.pallas_call(
    _ffn_body,
    grid_spec=_ffn_grid_spec,
    out_shape=jax.ShapeDtypeStruct((PADT, D), jnp.float32),
)


def _sc_mesh():
    return plsc.VectorSubcoreMesh(
        core_axis_name="c", subcore_axis_name="s",
        num_cores=NC, num_subcores=NS)


def _scatter_body(xf_hbm, dest_hbm, out_hbm, idx_v, rows_v,
                  s0, s1, s2, s3, so):
    wid = lax.axis_index("s") * NC + lax.axis_index("c")
    base = wid * RPW
    sems = (s0, s1, s2, s3)
    pltpu.sync_copy(dest_hbm.at[wid], idx_v)
    stages = [
        pltpu.async_copy(xf_hbm.at[pl.ds(base + k * CHK, CHK)],
                         rows_v.at[pl.ds(k * CHK, CHK)], sems[k])
        for k in range(NCHK)]
    outs = []
    for k in range(NCHK):
        stages[k].wait()
        outs.append(
            pltpu.async_copy(rows_v.at[pl.ds(k * CHK, CHK)],
                             out_hbm.at[idx_v.at[k]], so))
    for cp in outs:
        cp.wait()


def _gather_body(ys_hbm, dest_hbm, out_hbm, idx_v, rows_v,
                 s0, s1, s2, s3, so):
    wid = lax.axis_index("s") * NC + lax.axis_index("c")
    base = wid * RPW
    sems = (s0, s1, s2, s3)
    pltpu.sync_copy(dest_hbm.at[wid], idx_v)
    gets = [
        pltpu.async_copy(ys_hbm.at[idx_v.at[k]],
                         rows_v.at[pl.ds(k * CHK, CHK)], sems[k])
        for k in range(NCHK)]
    outs = []
    for k in range(NCHK):
        gets[k].wait()
        outs.append(
            pltpu.async_copy(rows_v.at[pl.ds(k * CHK, CHK)],
                             out_hbm.at[pl.ds(base + k * CHK, CHK)], so))
    for cp in outs:
        cp.wait()


def _sc_scatter_call(xf, dest):
    fn = pl.kernel(
        _scatter_body,
        out_type=jax.ShapeDtypeStruct((PADT, D), jnp.float32),
        mesh=_sc_mesh(),
        scratch_types=[
            pltpu.VMEM((NCHK, CHK), jnp.int32),
            pltpu.VMEM((RPW, D), jnp.float32),
            pltpu.SemaphoreType.DMA,
            pltpu.SemaphoreType.DMA,
            pltpu.SemaphoreType.DMA,
            pltpu.SemaphoreType.DMA,
            pltpu.SemaphoreType.DMA,
        ],
    )
    return fn(xf, dest)


def _sc_gather_call(ys, dest):
    fn = pl.kernel(
        _gather_body,
        out_type=jax.ShapeDtypeStruct((T, D), jnp.float32),
        mesh=_sc_mesh(),
        scratch_types=[
            pltpu.VMEM((NCHK, CHK), jnp.int32),
            pltpu.VMEM((RPW, D), jnp.float32),
            pltpu.SemaphoreType.DMA,
            pltpu.SemaphoreType.DMA,
            pltpu.SemaphoreType.DMA,
            pltpu.SemaphoreType.DMA,
            pltpu.SemaphoreType.DMA,
        ],
    )
    return fn(ys, dest)


def kernel(x, W_gate, b_gate, W_up, b_up, W_down, b_down):
    Bx, Tx, C = x.shape
    xf = x.reshape(T, D)
    gate, dest2, meta = _gate_call(xf, xf, xf, xf, W_gate,
                                   b_gate.reshape(1, E))
    dest = dest2.reshape(NW, NCHK, CHK)
    meta_f = meta.reshape(MAXB + 8)
    xs = _sc_scatter_call(xf, dest)
    wu_v = W_up.reshape(E, 2, D // 2, DFF)
    wd_v = W_down.reshape(E, 2, DFF // 2, D)
    ys = _ffn_call(meta_f, xs, wu_v, wu_v, b_up.reshape(E, 1, DFF),
                   wd_v, wd_v, b_down.reshape(E, 1, D))
    outf = _sc_gather_call(ys, dest)
    return outf.reshape(Bx, Tx, C), gate.reshape(Bx, Tx, E)


# speedup vs baseline: 1.0206x; 1.0206x over previous
import jax
import jax.numpy as jnp
from jax import lax
from jax.experimental import pallas as pl
from jax.experimental.pallas import tpu as pltpu
from jax.experimental.pallas import tpu_sc as plsc

D = 768
E = 16
DFF = 4 * D
T = 2048
BT = 256
MAXB = 24
PADT = MAXB * BT
TC = 256
NCH = T // TC
NC, NS = 2, 16
NW = NC * NS
RPW = T // NW


def _gate_body(x1_ref, x2_ref, x3_ref, x4_ref, wg_ref, bg_ref,
               gate_ref, dest_ref, meta_ref):
    wg = wg_ref[...]
    logits = jnp.concatenate(
        [jnp.dot(xr[...], wg, preferred_element_type=jnp.float32)
         for xr in (x1_ref, x2_ref, x3_ref, x4_ref)], axis=0) + bg_ref[...]
    lane = lax.broadcasted_iota(jnp.int32, (T, E), 1)
    maxv = jnp.max(logits, axis=1, keepdims=True)
    idx = jnp.min(jnp.where(logits == maxv, lane, E), axis=1, keepdims=True)
    oh_i = (lane == idx).astype(jnp.int32)
    oh_f = oh_i.astype(jnp.float32)
    gate_ref[...] = logits + (oh_f - logits)

    c = oh_i
    k = 1
    while k < T:
        c = c + jnp.concatenate(
            [jnp.zeros((k, E), jnp.int32), c[: T - k, :]], axis=0)
        k *= 2
    counts = c[T - 1 : T, :]
    nb = (counts + (BT - 1)) // BT
    be = nb
    k = 1
    while k < E:
        be = be + jnp.concatenate(
            [jnp.zeros((1, k), jnp.int32), be[:, : E - k]], axis=1)
        k *= 2
    pad_off = (be - nb) * BT
    total = be[:, E - 1 : E]

    biota = lax.broadcasted_iota(jnp.int32, (MAXB, E), 0)
    eob = jnp.sum((biota >= be).astype(jnp.int32), axis=1, keepdims=True)
    eob = jnp.minimum(eob, E - 1)
    meta_ref[...] = jnp.concatenate(
        [eob, total, jnp.zeros((7, 1), jnp.int32)], axis=0)

    dest_ref[...] = jnp.sum(oh_i * (pad_off + c - 1), axis=1, keepdims=True)


_gate_call = pl.pallas_call(
    _gate_body,
    grid=(1,),
    in_specs=(
        [pl.BlockSpec((T // 4, D), (lambda k: (lambda i: (k, 0)))(k))
         for k in range(4)]
        + [pl.BlockSpec((D, E), lambda i: (0, 0)),
           pl.BlockSpec((1, E), lambda i: (0, 0))]
    ),
    out_specs=[
        pl.BlockSpec((T, E), lambda i: (0, 0)),
        pl.BlockSpec((T, 1), lambda i: (0, 0)),
        pl.BlockSpec((MAXB + 8, 1), lambda i: (0, 0)),
    ],
    out_shape=(
        jax.ShapeDtypeStruct((T, E), jnp.float32),
        jax.ShapeDtypeStruct((T, 1), jnp.int32),
        jax.ShapeDtypeStruct((MAXB + 8, 1), jnp.int32),
    ),
)


def _ffn_body(meta_ref, x_ref, wu1_ref, wu2_ref, bu_ref, wd1_ref, wd2_ref,
              bd_ref, o_ref):
    b = pl.program_id(0)

    @pl.when(b < meta_ref[MAXB])
    def _():
        xb = x_ref[...].astype(jnp.bfloat16)
        h = (jnp.dot(xb[:, : D // 2], wu1_ref[0, 0].astype(jnp.bfloat16),
                     preferred_element_type=jnp.float32)
             + jnp.dot(xb[:, D // 2 :], wu2_ref[0, 0].astype(jnp.bfloat16),
                       preferred_element_type=jnp.float32)
             + bu_ref[0])
        h = jnp.maximum(h, 0.0).astype(jnp.bfloat16)
        o_ref[...] = (
            jnp.dot(h[:, : DFF // 2], wd1_ref[0, 0].astype(jnp.bfloat16),
                    preferred_element_type=jnp.float32)
            + jnp.dot(h[:, DFF // 2 :], wd2_ref[0, 0].astype(jnp.bfloat16),
                      preferred_element_type=jnp.float32)
            + bd_ref[0])


def _bmap(b, m):
    return (jnp.minimum(b, m[MAXB] - 1), 0)


def _emap3(b, m):
    return (m[jnp.minimum(b, m[MAXB] - 1)], 0, 0)


def _emap4a(b, m):
    return (m[jnp.minimum(b, m[MAXB] - 1)], 0, 0, 0)


def _emap4b(b, m):
    return (m[jnp.minimum(b, m[MAXB] - 1)], 1, 0, 0)


_ffn_grid_spec = pltpu.PrefetchScalarGridSpec(
    num_scalar_prefetch=1,
    grid=(MAXB,),
    in_specs=[
        pl.BlockSpec((BT, D), _bmap),
        pl.BlockSpec((1, 1, D // 2, DFF), _emap4a),
        pl.BlockSpec((1, 1, D // 2, DFF), _emap4b),
        pl.BlockSpec((1, 1, DFF), _emap3),
        pl.BlockSpec((1, 1, DFF // 2, D), _emap4a),
        pl.BlockSpec((1, 1, DFF // 2, D), _emap4b),
        pl.BlockSpec((1, 1, D), _emap3),
    ],
    out_specs=pl.BlockSpec((BT, D), _bmap),
)

_ffn_call = pl.pallas_call(
    _ffn_body,
    grid_spec=_ffn_grid_spec,
    out_shape=jax.ShapeDtypeStruct((PADT, D), jnp.float32),
)


def _sc_mesh():
    return plsc.VectorSubcoreMesh(
        core_axis_name="c", subcore_axis_name="s",
        num_cores=NC, num_subcores=NS)


def _scatter_body(xf_hbm, dest_hbm, out_hbm, idx_v, rows_v, sem):
    wid = lax.axis_index("s") * NC + lax.axis_index("c")
    base = wid * RPW
    pltpu.sync_copy(dest_hbm.at[pl.ds(base, RPW)], idx_v)
    pltpu.sync_copy(xf_hbm.at[pl.ds(base, RPW)], rows_v)
    pltpu.async_copy(rows_v, out_hbm.at[idx_v], sem).wait()


def _gather_body(ys_hbm, dest_hbm, out_hbm, idx_v, rows_v, sem):
    wid = lax.axis_index("s") * NC + lax.axis_index("c")
    base = wid * RPW
    pltpu.sync_copy(dest_hbm.at[pl.ds(base, RPW)], idx_v)
    pltpu.async_copy(ys_hbm.at[idx_v], rows_v, sem).wait()
    pltpu.sync_copy(rows_v, out_hbm.at[pl.ds(base, RPW)])


def _sc_scatter_call(xf, dest):
    fn = pl.kernel(
        _scatter_body,
        out_type=jax.ShapeDtypeStruct((PADT, D), jnp.float32),
        mesh=_sc_mesh(),
        scratch_types=[
            pltpu.VMEM((RPW,), jnp.int32),
            pltpu.VMEM((RPW, D), jnp.float32),
            pltpu.SemaphoreType.DMA,
        ],
    )
    return fn(xf, dest)


def _sc_gather_call(ys, dest):
    fn = pl.kernel(
        _gather_body,
        out_type=jax.ShapeDtypeStruct((T, D), jnp.float32),
        mesh=_sc_mesh(),
        scratch_types=[
            pltpu.VMEM((RPW,), jnp.int32),
            pltpu.VMEM((RPW, D), jnp.float32),
            pltpu.SemaphoreType.DMA,
        ],
    )
    return fn(ys, dest)


def kernel(x, W_gate, b_gate, W_up, b_up, W_down, b_down):
    Bx, Tx, C = x.shape
    xf = x.reshape(T, D)
    gate, dest2, meta = _gate_call(xf, xf, xf, xf, W_gate,
                                   b_gate.reshape(1, E))
    dest = dest2.reshape(T)
    meta_f = meta.reshape(MAXB + 8)
    xs = _sc_scatter_call(xf, dest)
    wu_v = W_up.reshape(E, 2, D // 2, DFF)
    wd_v = W_down.reshape(E, 2, DFF // 2, D)
    ys = _ffn_call(meta_f, xs, wu_v, wu_v, b_up.reshape(E, 1, DFF),
                   wd_v, wd_v, b_down.reshape(E, 1, D))
    outf = _sc_gather_call(ys, dest)
    return outf.reshape(Bx, Tx, C), gate.reshape(Bx, Tx, E)
